# baseline (speedup 1.0000x reference)
import functools

import jax
import jax.numpy as jnp
from jax import lax
from jax.experimental import pallas as pl
from jax.experimental.pallas import tpu as pltpu
import jax.experimental.pallas.tpu_sc as plsc

N = 10000
E = 320000
G = 16
DN = 128
DE = 16
DU = 32
NCLS = 10
NPASS = 3

NC = 2
NS = 16
NW = NC * NS
EPW = E // NW
SUB = 125
NSUB = 5
CH = SUB * NSUB
NCHUNK = EPW // CH
NP = 10240
RPT = NP // NS
SPT = N // NS

_f32 = jnp.float32


def _oh(b2):
    return (b2[:] == lax.broadcasted_iota(jnp.int32, (1, G), 1)).astype(_f32)


def _prep_body(na, b2, u, wes, wed, weu, be, e8, bd, p_o, pd_o, q_o):
    oh = _oh(b2)
    ub = jnp.dot(u[:], weu[:], preferred_element_type=_f32)
    p_o[:] = (jnp.dot(na[:], wes[:], preferred_element_type=_f32)
              + jnp.dot(oh, ub, preferred_element_type=_f32) + be[:])
    pd_o[:] = jnp.dot(na[:], wed[:], preferred_element_type=_f32)
    q_o[:] = jnp.dot(e8[:], bd[:], preferred_element_type=_f32)


def _q_body(e8, bd, q8):
    q8[:] = jnp.dot(e8[:], bd[:], preferred_element_type=_f32)


def _node_impl(na, s2, invdeg, b2, u, wnn, wne, wnu, bn, wgn, wge, wgu, bg):
    oh = _oh(b2)
    eob = (s2[0, :N] + s2[1, :N]) * invdeg
    ub = jnp.dot(u[:], wnu[:], preferred_element_type=_f32)
    x = jnp.dot(na[:], wnn[:], preferred_element_type=_f32)
    x = x + jnp.dot(eob, wne[:], preferred_element_type=_f32)
    x = x + jnp.dot(oh, ub, preferred_element_type=_f32) + bn[:]
    x = jnp.maximum(x, 0.0)
    dn = (((0,), (0,)), ((), ()))
    nbar = lax.dot_general(oh, x, dn, preferred_element_type=_f32)
    ebar = lax.dot_general(oh, eob, dn, preferred_element_type=_f32)
    cnt = lax.dot_general(oh, jnp.ones((N, 1), _f32), dn,
                          preferred_element_type=_f32)
    c = jnp.maximum(cnt, 1.0)
    nm = nbar / c
    em = ebar / c
    un = jnp.dot(nm, wgn[:], preferred_element_type=_f32)
    un = un + jnp.dot(em, wge[:], preferred_element_type=_f32)
    un = un + jnp.dot(u[:], wgu[:], preferred_element_type=_f32) + bg[:]
    un = jnp.maximum(un, 0.0)
    return oh, x, un, nm, em, cnt


def _node1_body(na, s2, c2, b2, u, wnn, wne, wnu, bn, wgn, wge, wgu, bg,
                wes, wed, weu, be,
                na_o, u_o, p_o, pd_o, inv_o):
    invdeg = 1.0 / jnp.maximum(c2[0, :N] + c2[1, :N], 1.0)
    inv_o[:] = invdeg
    oh, x, un, nm, em, cnt = _node_impl(
        na, s2, invdeg, b2, u, wnn, wne, wnu, bn, wgn, wge, wgu, bg)
    na_o[:] = x
    u_o[:] = un
    ub = jnp.dot(un, weu[:], preferred_element_type=_f32)
    p_o[:] = (jnp.dot(x, wes[:], preferred_element_type=_f32)
              + jnp.dot(oh, ub, preferred_element_type=_f32) + be[:])
    pd_o[:] = jnp.dot(x, wed[:], preferred_element_type=_f32)


def _node2_body(na, s2, inv_i, b2, u, wnn, wne, wnu, bn, wgn, wge, wgu, bg,
                wes, wed, weu, be,
                na_o, u_o, p_o, pd_o):
    oh, x, un, nm, em, cnt = _node_impl(
        na, s2, inv_i[:], b2, u, wnn, wne, wnu, bn, wgn, wge, wgu, bg)
    na_o[:] = x
    u_o[:] = un
    ub = jnp.dot(un, weu[:], preferred_element_type=_f32)
    p_o[:] = (jnp.dot(x, wes[:], preferred_element_type=_f32)
              + jnp.dot(oh, ub, preferred_element_type=_f32) + be[:])
    pd_o[:] = jnp.dot(x, wed[:], preferred_element_type=_f32)


def _node3_body(na, s2, inv_i, b2, u, wnn, wne, wnu, bn, wgn, wge, wgu, bg,
                wln, wle, wlu, bl,
                out_o):
    oh, x, un, nm, em, cnt = _node_impl(
        na, s2, inv_i[:], b2, u, wnn, wne, wnu, bn, wgn, wge, wgu, bg)
    um = un * (cnt > 0.0).astype(_f32)
    y = jnp.dot(nm, wln[:], preferred_element_type=_f32)
    y = y + jnp.dot(em, wle[:], preferred_element_type=_f32)
    y = y + jnp.dot(um, wlu[:], preferred_element_type=_f32) + bl[:]
    out_o[:] = y


def _sc_edge_impl(src_h, dst_h, p_h, pd_h, q_h, eo_h, s_h, cnt_h,
                  sidx, didx, pb, pdb, qb, ob, zb,
                  acc, sem_i, sem_g, sem_s, sem_t,
                  ones_b, cacc):
    c = lax.axis_index("c")
    s = lax.axis_index("s")
    w = c * NS + s

    def zrow(i, _):
        zb[i] = jnp.zeros((DE,), _f32)
        return 0
    lax.fori_loop(0, RPT, zrow, 0)
    pltpu.sync_copy(zb, acc.at[pl.ds(s * RPT, RPT)])
    if cnt_h is not None:
        pltpu.sync_copy(zb, cacc.at[pl.ds(s * RPT, RPT)])

        def orow(i, _):
            ones_b[i] = jnp.ones((DE,), _f32)
            return 0
        lax.fori_loop(0, SUB, orow, 0)
    plsc.subcore_barrier()

    idx_row0 = w * (EPW // SUB)

    def load_idx(j):
        r0 = idx_row0 + j * NSUB
        b = j % 2
        pltpu.sync_copy(src_h.at[pl.ds(r0, NSUB)], sidx[b])
        pltpu.sync_copy(dst_h.at[pl.ds(r0, NSUB)], didx[b])

    def fire_gathers(j):
        b = j % 2
        cps = []
        for k in range(NSUB):
            cps.append(pltpu.async_copy(
                p_h.at[sidx[b].at[k]], pb[b].at[pl.ds(k * SUB, SUB)],
                sem_g[b]))
            cps.append(pltpu.async_copy(
                pd_h.at[didx[b].at[k]], pdb[b].at[pl.ds(k * SUB, SUB)],
                sem_g[b]))
        cps.append(pltpu.async_copy(
            q_h.at[pl.ds(w * EPW + j * CH, CH)], qb[b], sem_g[b]))
        return cps

    g_d = [None, None]
    load_idx(0)
    g_d[0] = fire_gathers(0)
    for j in range(NCHUNK):
        b = j % 2
        if j + 1 < NCHUNK:
            load_idx(j + 1)
            g_d[(j + 1) % 2] = fire_gathers(j + 1)
        for d in g_d[b]:
            d.wait()

        pbj, pdj, qj = pb[b], pdb[b], qb[b]

        def erow(i, _, pbj=pbj, pdj=pdj, qj=qj):
            ob[0][i] = jnp.maximum(pbj[i] + pdj[i] + qj[i], 0.0)
            return 0
        lax.fori_loop(0, CH, erow, 0)

        e0 = w * EPW + j * CH
        pltpu.sync_copy(ob[0], eo_h.at[pl.ds(e0, CH)])
        for k in range(NSUB):
            pltpu.sync_copy(ob[0].at[pl.ds(k * SUB, SUB)],
                            acc.at[didx[b].at[k]], add=True)
            if cnt_h is not None:
                pltpu.sync_copy(ones_b, cacc.at[didx[b].at[k]], add=True)

    plsc.subcore_barrier()

    r = s * RPT
    pltpu.sync_copy(acc.at[pl.ds(r, RPT)], zb)
    pltpu.sync_copy(zb, s_h.at[c, pl.ds(r, RPT)])
    if cnt_h is not None:
        pltpu.sync_copy(cacc.at[pl.ds(r, RPT)], zb)
        pltpu.sync_copy(zb, cnt_h.at[c, pl.ds(r, RPT)])


def _make_sc(with_counts):
    outs = [jax.ShapeDtypeStruct((E, DE), _f32),
            jax.ShapeDtypeStruct((NC, NP, DE), _f32)]
    scratch = (
        [pltpu.VMEM((NSUB, SUB), jnp.int32) for _ in range(4)]
        + [pltpu.VMEM((NSUB, SUB), jnp.int32) for _ in range(4)]
        + [pltpu.VMEM((CH, DE), _f32) for _ in range(2)]
        + [pltpu.VMEM((CH, DE), _f32) for _ in range(2)]
        + [pltpu.VMEM((CH, DE), _f32) for _ in range(2)]
        + [pltpu.VMEM((CH, DE), _f32) for _ in range(2)]
        + [pltpu.VMEM((RPT, DE), _f32)]
        + [pltpu.VMEM_SHARED((NP, DE), _f32)]
        + [pltpu.SemaphoreType.DMA for _ in range(4)]
        + [pltpu.SemaphoreType.DMA for _ in range(2)]
        + [pltpu.SemaphoreType.DMA for _ in range(2)]
        + [pltpu.SemaphoreType.DMA]
    )
    if with_counts:
        outs.append(jax.ShapeDtypeStruct((NC, NP, DE), _f32))
        scratch.append(pltpu.VMEM((SUB, DE), _f32))
        scratch.append(pltpu.VMEM_SHARED((NP, DE), _f32))
    mesh = plsc.VectorSubcoreMesh(core_axis_name="c", subcore_axis_name="s")
    cparams = pltpu.CompilerParams(use_tc_tiling_on_sc=False)

    if with_counts:
        @functools.partial(pl.kernel, out_type=outs, mesh=mesh,
                           scratch_types=scratch, compiler_params=cparams)
        def k(src_h, dst_h, p_h, pd_h, q_h, eo_h, s_h, cnt_h,
              si0, si1, si2, si3, di0, di1, di2, di3,
              pb0, pb1, pdb0, pdb1, qb0, qb1, ob0, ob1, zb,
              acc, smi0, smi1, smi2, smi3, smg0, smg1,
              sms0, sms1, smt, ones_b, cacc):
            _sc_edge_impl(src_h, dst_h, p_h, pd_h, q_h, eo_h, s_h, cnt_h,
                          [si0, si1, si2, si3], [di0, di1, di2, di3],
                          [pb0, pb1], [pdb0, pdb1], [qb0, qb1], [ob0, ob1],
                          zb, acc,
                          [smi0, smi1, smi2, smi3], [smg0, smg1],
                          [sms0, sms1], smt, ones_b, cacc)
    else:
        @functools.partial(pl.kernel, out_type=outs, mesh=mesh,
                           scratch_types=scratch, compiler_params=cparams)
        def k(src_h, dst_h, p_h, pd_h, q_h, eo_h, s_h,
              si0, si1, si2, si3, di0, di1, di2, di3,
              pb0, pb1, pdb0, pdb1, qb0, qb1, ob0, ob1, zb,
              acc, smi0, smi1, smi2, smi3, smg0, smg1,
              sms0, sms1, smt):
            _sc_edge_impl(src_h, dst_h, p_h, pd_h, q_h, eo_h, s_h, None,
                          [si0, si1, si2, si3], [di0, di1, di2, di3],
                          [pb0, pb1], [pdb0, pdb1], [qb0, qb1], [ob0, ob1],
                          zb, acc,
                          [smi0, smi1, smi2, smi3], [smg0, smg1],
                          [sms0, sms1], smt, None, None)
    return k


@functools.lru_cache(maxsize=None)
def _get_sc(with_counts):
    return _make_sc(with_counts)


_VMEM_BIG = pltpu.CompilerParams(vmem_limit_bytes=100 * 1024 * 1024)


def _prep_call(na, b2, u, wes, wed, weu, be1, e8, bd):
    return pl.pallas_call(
        _prep_body,
        compiler_params=_VMEM_BIG,
        out_shape=[jax.ShapeDtypeStruct((N, DE), _f32),
                   jax.ShapeDtypeStruct((N, DE), _f32),
                   jax.ShapeDtypeStruct((E // 8, 8 * DE), _f32)],
    )(na, b2, u, wes, wed, weu, be1, e8, bd)


def _q_call(e8, bd):
    e8r = E // 8
    blk = e8r // 10
    return pl.pallas_call(
        _q_body,
        grid=(10,),
        in_specs=[pl.BlockSpec((blk, 8 * DE), lambda i: (i, 0)),
                  pl.BlockSpec((8 * DE, 8 * DE), lambda i: (0, 0))],
        out_specs=pl.BlockSpec((blk, 8 * DE), lambda i: (i, 0)),
        out_shape=jax.ShapeDtypeStruct((e8r, 8 * DE), _f32),
    )(e8, bd)


def _node1_call(na, s2, c2, b2, u, ws):
    return pl.pallas_call(
        _node1_body,
        compiler_params=_VMEM_BIG,
        out_shape=[jax.ShapeDtypeStruct((N, DN), _f32),
                   jax.ShapeDtypeStruct((G, DU), _f32),
                   jax.ShapeDtypeStruct((N, DE), _f32),
                   jax.ShapeDtypeStruct((N, DE), _f32),
                   jax.ShapeDtypeStruct((N, DE), _f32)],
    )(na, s2, c2, b2, u, *ws)


def _node2_call(na, s2, inv, b2, u, ws):
    return pl.pallas_call(
        _node2_body,
        compiler_params=_VMEM_BIG,
        out_shape=[jax.ShapeDtypeStruct((N, DN), _f32),
                   jax.ShapeDtypeStruct((G, DU), _f32),
                   jax.ShapeDtypeStruct((N, DE), _f32),
                   jax.ShapeDtypeStruct((N, DE), _f32)],
    )(na, s2, inv, b2, u, *ws)


def _node3_call(na, s2, inv, b2, u, ws):
    return pl.pallas_call(
        _node3_body,
        compiler_params=_VMEM_BIG,
        out_shape=jax.ShapeDtypeStruct((G, NCLS), _f32),
    )(na, s2, inv, b2, u, *ws)


def kernel(node_attr, edge_attr, u, edge_index, batch,
           We, be, Wn, bn, Wg, bg, Wl, bl):
    src2 = edge_index[0].reshape(E // SUB, SUB)
    dst2 = edge_index[1].reshape(E // SUB, SUB)
    b2 = batch.reshape(N, 1)

    wes, wed, wee, weu = (We[:DN], We[DN:2 * DN],
                          We[2 * DN:2 * DN + DE], We[2 * DN + DE:])
    wnn, wne, wnu = Wn[:DN], Wn[DN:DN + DE], Wn[DN + DE:]
    wgn, wge, wgu = Wg[:DN], Wg[DN:DN + DE], Wg[DN + DE:]
    wln, wle, wlu = Wl[:DN], Wl[DN:DN + DE], Wl[DN + DE:]
    be1 = be.reshape(1, DE)
    bn1 = bn.reshape(1, DN)
    bg1 = bg.reshape(1, DU)
    bl1 = bl.reshape(1, NCLS)
    bd = jnp.kron(jnp.eye(8, dtype=_f32), wee)

    w_mid = (wnn, wne, wnu, bn1, wgn, wge, wgu, bg1, wes, wed, weu, be1)
    w_fin = (wnn, wne, wnu, bn1, wgn, wge, wgu, bg1, wln, wle, wlu, bl1)

    p, pd, q8 = _prep_call(node_attr, b2, u, wes, wed, weu, be1,
                           edge_attr.reshape(E // 8, 8 * DE), bd)
    na = node_attr
    inv = None
    for ps in range(NPASS):
        if ps > 0:
            q8 = _q_call(e8, bd)
        q = q8.reshape(E, DE)
        if ps == 0:
            edge, s2, cnt2 = _get_sc(True)(src2, dst2, p, pd, q)
        else:
            edge, s2 = _get_sc(False)(src2, dst2, p, pd, q)
        e8 = edge.reshape(E // 8, 8 * DE)
        if ps == 0:
            na, u, p, pd, inv = _node1_call(na, s2, cnt2, b2, u, w_mid)
        elif ps < NPASS - 1:
            na, u, p, pd = _node2_call(na, s2, inv, b2, u, w_mid)
        else:
            return _node3_call(na, s2, inv, b2, u, w_fin)


# speedup vs baseline: 1.0674x; 1.0674x over previous
import functools

import jax
import jax.numpy as jnp
from jax import lax
from jax.experimental import pallas as pl
from jax.experimental.pallas import tpu as pltpu
import jax.experimental.pallas.tpu_sc as plsc

N = 10000
E = 320000
G = 16
DN = 128
DE = 16
DU = 32
NCLS = 10
NPASS = 3

NC = 2
NS = 16
NW = NC * NS
EPW = E // NW
SUB = 125
NSUB = 5
CH = SUB * NSUB
NCHUNK = EPW // CH
NP = 10240
RPT = NP // NS
SPT = N // NS

_f32 = jnp.float32


def _oh(b2):
    return (b2[:] == lax.broadcasted_iota(jnp.int32, (1, G), 1)).astype(_f32)


def _prep_body(na, b2, u, wes, wed, weu, be, e8, bd, p_o, pd_o, q_o):
    oh = _oh(b2)
    ub = jnp.dot(u[:], weu[:], preferred_element_type=_f32)
    p_o[:] = (jnp.dot(na[:], wes[:], preferred_element_type=_f32)
              + jnp.dot(oh, ub, preferred_element_type=_f32) + be[:])
    pd_o[:] = jnp.dot(na[:], wed[:], preferred_element_type=_f32)
    q_o[:] = jnp.dot(e8[:], bd[:], preferred_element_type=_f32)


def _q_body(e8, bd, q8):
    q8[:] = jnp.dot(e8[:], bd[:], preferred_element_type=_f32)


def _node_impl(na, s2r, invr, br, b2, u, wnn, kne, wnu, bn,
               wgn, wge, wgu, bg):
    oh = _oh(b2)
    z = (s2r[0] + s2r[1]) * invr
    ub = jnp.dot(u[:], wnu[:], preferred_element_type=_f32)
    x = jnp.dot(na[:], wnn[:], preferred_element_type=_f32)
    y = jnp.dot(z, kne[:], preferred_element_type=_f32)
    y = jnp.reshape(y, (NP, DN))[:N]
    x = x + y
    x = x + jnp.dot(oh, ub, preferred_element_type=_f32) + bn[:]
    x = jnp.maximum(x, 0.0)
    dn = (((0,), (0,)), ((), ()))
    nbar = lax.dot_general(oh, x, dn, preferred_element_type=_f32)
    ebar = jnp.zeros((G, DE), _f32)
    for k in range(8):
        ohk = (br[:, k:k + 1] == lax.broadcasted_iota(jnp.int32, (1, G), 1)
               ).astype(_f32)
        ebar = ebar + lax.dot_general(ohk, z[:, k * DE:(k + 1) * DE], dn,
                                      preferred_element_type=_f32)
    cnt = lax.dot_general(oh, jnp.ones((N, 1), _f32), dn,
                          preferred_element_type=_f32)
    c = jnp.maximum(cnt, 1.0)
    nm = nbar / c
    em = ebar / c
    un = jnp.dot(nm, wgn[:], preferred_element_type=_f32)
    un = un + jnp.dot(em, wge[:], preferred_element_type=_f32)
    un = un + jnp.dot(u[:], wgu[:], preferred_element_type=_f32) + bg[:]
    un = jnp.maximum(un, 0.0)
    return oh, x, un, nm, em, cnt


def _pp_out(oh, x, un, wes, wed, weu, be, p_o, pd_o):
    ub = jnp.dot(un, weu[:], preferred_element_type=_f32)
    p_o[:] = (jnp.dot(x, wes[:], preferred_element_type=_f32)
              + jnp.dot(oh, ub, preferred_element_type=_f32) + be[:])
    pd_o[:] = jnp.dot(x, wed[:], preferred_element_type=_f32)


def _node1_body(na, s2r, c2r, br, b2, u, wnn, kne, wnu, bn,
                wgn, wge, wgu, bg, wes, wed, weu, be,
                na_o, u_o, p_o, pd_o, inv_o):
    invr = 1.0 / jnp.maximum(c2r[0] + c2r[1], 1.0)
    inv_o[:] = invr
    oh, x, un, nm, em, cnt = _node_impl(
        na, s2r, invr, br, b2, u, wnn, kne, wnu, bn, wgn, wge, wgu, bg)
    na_o[:] = x
    u_o[:] = un
    _pp_out(oh, x, un, wes, wed, weu, be, p_o, pd_o)


def _node2_body(na, s2r, inv_i, br, b2, u, wnn, kne, wnu, bn,
                wgn, wge, wgu, bg, wes, wed, weu, be,
                na_o, u_o, p_o, pd_o):
    oh, x, un, nm, em, cnt = _node_impl(
        na, s2r, inv_i[:], br, b2, u, wnn, kne, wnu, bn, wgn, wge, wgu, bg)
    na_o[:] = x
    u_o[:] = un
    _pp_out(oh, x, un, wes, wed, weu, be, p_o, pd_o)


def _node3_body(na, s2r, inv_i, br, b2, u, wnn, kne, wnu, bn,
                wgn, wge, wgu, bg, wln, wle, wlu, bl,
                out_o):
    oh, x, un, nm, em, cnt = _node_impl(
        na, s2r, inv_i[:], br, b2, u, wnn, kne, wnu, bn, wgn, wge, wgu, bg)
    um = un * (cnt > 0.0).astype(_f32)
    y = jnp.dot(nm, wln[:], preferred_element_type=_f32)
    y = y + jnp.dot(em, wle[:], preferred_element_type=_f32)
    y = y + jnp.dot(um, wlu[:], preferred_element_type=_f32) + bl[:]
    out_o[:] = y


def _sc_edge_impl(src_h, dst_h, p_h, pd_h, q_h, eo_h, s_h, cnt_h,
                  sidx, didx, pb, pdb, qb, ob, zb,
                  acc, sem_i, sem_g, sem_s, sem_t,
                  ones_b, cacc):
    c = lax.axis_index("c")
    s = lax.axis_index("s")
    w = c * NS + s

    def zrow(i, _):
        zb[i] = jnp.zeros((DE,), _f32)
        return 0
    lax.fori_loop(0, RPT, zrow, 0)
    pltpu.sync_copy(zb, acc.at[pl.ds(s * RPT, RPT)])
    if cnt_h is not None:
        pltpu.sync_copy(zb, cacc.at[pl.ds(s * RPT, RPT)])

        def orow(i, _):
            ones_b[i] = jnp.ones((DE,), _f32)
            return 0
        lax.fori_loop(0, SUB, orow, 0)
    plsc.subcore_barrier()

    idx_row0 = w * (EPW // SUB)

    def load_idx(j):
        r0 = idx_row0 + j * NSUB
        b = j % 2
        pltpu.sync_copy(src_h.at[pl.ds(r0, NSUB)], sidx[b])
        pltpu.sync_copy(dst_h.at[pl.ds(r0, NSUB)], didx[b])

    def fire_gathers(j):
        b = j % 2
        cps = []
        for k in range(NSUB):
            cps.append(pltpu.async_copy(
                p_h.at[sidx[b].at[k]], pb[b].at[pl.ds(k * SUB, SUB)],
                sem_g[b]))
            cps.append(pltpu.async_copy(
                pd_h.at[didx[b].at[k]], pdb[b].at[pl.ds(k * SUB, SUB)],
                sem_g[b]))
        cps.append(pltpu.async_copy(
            q_h.at[pl.ds(w * EPW + j * CH, CH)], qb[b], sem_g[b]))
        return cps

    g_d = [None, None]
    load_idx(0)
    g_d[0] = fire_gathers(0)
    for j in range(NCHUNK):
        b = j % 2
        if j + 1 < NCHUNK:
            load_idx(j + 1)
            g_d[(j + 1) % 2] = fire_gathers(j + 1)
        for d in g_d[b]:
            d.wait()

        pbj, pdj, qj = pb[b], pdb[b], qb[b]

        def erow(i, _, pbj=pbj, pdj=pdj, qj=qj):
            ob[0][i] = jnp.maximum(pbj[i] + pdj[i] + qj[i], 0.0)
            return 0
        lax.fori_loop(0, CH, erow, 0)

        e0 = w * EPW + j * CH
        pltpu.sync_copy(ob[0], eo_h.at[pl.ds(e0, CH)])
        for k in range(NSUB):
            pltpu.sync_copy(ob[0].at[pl.ds(k * SUB, SUB)],
                            acc.at[didx[b].at[k]], add=True)
            if cnt_h is not None:
                pltpu.sync_copy(ones_b, cacc.at[didx[b].at[k]], add=True)

    plsc.subcore_barrier()

    r = s * RPT
    pltpu.sync_copy(acc.at[pl.ds(r, RPT)], zb)
    pltpu.sync_copy(zb, s_h.at[c, pl.ds(r, RPT)])
    if cnt_h is not None:
        pltpu.sync_copy(cacc.at[pl.ds(r, RPT)], zb)
        pltpu.sync_copy(zb, cnt_h.at[c, pl.ds(r, RPT)])


def _make_sc(with_counts):
    outs = [jax.ShapeDtypeStruct((E, DE), _f32),
            jax.ShapeDtypeStruct((NC, NP, DE), _f32)]
    scratch = (
        [pltpu.VMEM((NSUB, SUB), jnp.int32) for _ in range(4)]
        + [pltpu.VMEM((NSUB, SUB), jnp.int32) for _ in range(4)]
        + [pltpu.VMEM((CH, DE), _f32) for _ in range(2)]
        + [pltpu.VMEM((CH, DE), _f32) for _ in range(2)]
        + [pltpu.VMEM((CH, DE), _f32) for _ in range(2)]
        + [pltpu.VMEM((CH, DE), _f32) for _ in range(2)]
        + [pltpu.VMEM((RPT, DE), _f32)]
        + [pltpu.VMEM_SHARED((NP, DE), _f32)]
        + [pltpu.SemaphoreType.DMA for _ in range(4)]
        + [pltpu.SemaphoreType.DMA for _ in range(2)]
        + [pltpu.SemaphoreType.DMA for _ in range(2)]
        + [pltpu.SemaphoreType.DMA]
    )
    if with_counts:
        outs.append(jax.ShapeDtypeStruct((NC, NP, DE), _f32))
        scratch.append(pltpu.VMEM((SUB, DE), _f32))
        scratch.append(pltpu.VMEM_SHARED((NP, DE), _f32))
    mesh = plsc.VectorSubcoreMesh(core_axis_name="c", subcore_axis_name="s")
    cparams = pltpu.CompilerParams(use_tc_tiling_on_sc=False)

    if with_counts:
        @functools.partial(pl.kernel, out_type=outs, mesh=mesh,
                           scratch_types=scratch, compiler_params=cparams)
        def k(src_h, dst_h, p_h, pd_h, q_h, eo_h, s_h, cnt_h,
              si0, si1, si2, si3, di0, di1, di2, di3,
              pb0, pb1, pdb0, pdb1, qb0, qb1, ob0, ob1, zb,
              acc, smi0, smi1, smi2, smi3, smg0, smg1,
              sms0, sms1, smt, ones_b, cacc):
            _sc_edge_impl(src_h, dst_h, p_h, pd_h, q_h, eo_h, s_h, cnt_h,
                          [si0, si1, si2, si3], [di0, di1, di2, di3],
                          [pb0, pb1], [pdb0, pdb1], [qb0, qb1], [ob0, ob1],
                          zb, acc,
                          [smi0, smi1, smi2, smi3], [smg0, smg1],
                          [sms0, sms1], smt, ones_b, cacc)
    else:
        @functools.partial(pl.kernel, out_type=outs, mesh=mesh,
                           scratch_types=scratch, compiler_params=cparams)
        def k(src_h, dst_h, p_h, pd_h, q_h, eo_h, s_h,
              si0, si1, si2, si3, di0, di1, di2, di3,
              pb0, pb1, pdb0, pdb1, qb0, qb1, ob0, ob1, zb,
              acc, smi0, smi1, smi2, smi3, smg0, smg1,
              sms0, sms1, smt):
            _sc_edge_impl(src_h, dst_h, p_h, pd_h, q_h, eo_h, s_h, None,
                          [si0, si1, si2, si3], [di0, di1, di2, di3],
                          [pb0, pb1], [pdb0, pdb1], [qb0, qb1], [ob0, ob1],
                          zb, acc,
                          [smi0, smi1, smi2, smi3], [smg0, smg1],
                          [sms0, sms1], smt, None, None)
    return k


@functools.lru_cache(maxsize=None)
def _get_sc(with_counts):
    return _make_sc(with_counts)


_VMEM_BIG = pltpu.CompilerParams(vmem_limit_bytes=100 * 1024 * 1024)


def _prep_call(na, b2, u, wes, wed, weu, be1, e8, bd):
    return pl.pallas_call(
        _prep_body,
        compiler_params=_VMEM_BIG,
        out_shape=[jax.ShapeDtypeStruct((N, DE), _f32),
                   jax.ShapeDtypeStruct((N, DE), _f32),
                   jax.ShapeDtypeStruct((E // 8, 8 * DE), _f32)],
    )(na, b2, u, wes, wed, weu, be1, e8, bd)


def _q_call(e8, bd):
    e8r = E // 8
    blk = e8r // 10
    return pl.pallas_call(
        _q_body,
        grid=(10,),
        in_specs=[pl.BlockSpec((blk, 8 * DE), lambda i: (i, 0)),
                  pl.BlockSpec((8 * DE, 8 * DE), lambda i: (0, 0))],
        out_specs=pl.BlockSpec((blk, 8 * DE), lambda i: (i, 0)),
        out_shape=jax.ShapeDtypeStruct((e8r, 8 * DE), _f32),
    )(e8, bd)


_PK = NP * DE // 128


def _node1_call(na, s2r, c2r, br, b2, u, ws):
    return pl.pallas_call(
        _node1_body,
        compiler_params=_VMEM_BIG,
        out_shape=[jax.ShapeDtypeStruct((N, DN), _f32),
                   jax.ShapeDtypeStruct((G, DU), _f32),
                   jax.ShapeDtypeStruct((N, DE), _f32),
                   jax.ShapeDtypeStruct((N, DE), _f32),
                   jax.ShapeDtypeStruct((_PK, 128), _f32)],
    )(na, s2r, c2r, br, b2, u, *ws)


def _node2_call(na, s2r, inv, br, b2, u, ws):
    return pl.pallas_call(
        _node2_body,
        compiler_params=_VMEM_BIG,
        out_shape=[jax.ShapeDtypeStruct((N, DN), _f32),
                   jax.ShapeDtypeStruct((G, DU), _f32),
                   jax.ShapeDtypeStruct((N, DE), _f32),
                   jax.ShapeDtypeStruct((N, DE), _f32)],
    )(na, s2r, inv, br, b2, u, *ws)


def _node3_call(na, s2r, inv, br, b2, u, ws):
    return pl.pallas_call(
        _node3_body,
        compiler_params=_VMEM_BIG,
        out_shape=jax.ShapeDtypeStruct((G, NCLS), _f32),
    )(na, s2r, inv, br, b2, u, *ws)


def kernel(node_attr, edge_attr, u, edge_index, batch,
           We, be, Wn, bn, Wg, bg, Wl, bl):
    src2 = edge_index[0].reshape(E // SUB, SUB)
    dst2 = edge_index[1].reshape(E // SUB, SUB)
    b2 = batch.reshape(N, 1)
    br = jnp.pad(batch, (0, NP - N), constant_values=G).reshape(NP // 8, 8)

    wes, wed, wee, weu = (We[:DN], We[DN:2 * DN],
                          We[2 * DN:2 * DN + DE], We[2 * DN + DE:])
    wnn, wne, wnu = Wn[:DN], Wn[DN:DN + DE], Wn[DN + DE:]
    wgn, wge, wgu = Wg[:DN], Wg[DN:DN + DE], Wg[DN + DE:]
    wln, wle, wlu = Wl[:DN], Wl[DN:DN + DE], Wl[DN + DE:]
    be1 = be.reshape(1, DE)
    bn1 = bn.reshape(1, DN)
    bg1 = bg.reshape(1, DU)
    bl1 = bl.reshape(1, NCLS)
    bd = jnp.kron(jnp.eye(8, dtype=_f32), wee)
    kne = jnp.kron(jnp.eye(8, dtype=_f32), wne)

    w_mid = (wnn, kne, wnu, bn1, wgn, wge, wgu, bg1, wes, wed, weu, be1)
    w_fin = (wnn, kne, wnu, bn1, wgn, wge, wgu, bg1, wln, wle, wlu, bl1)

    p, pd, q8 = _prep_call(node_attr, b2, u, wes, wed, weu, be1,
                           edge_attr.reshape(E // 8, 8 * DE), bd)
    na = node_attr
    inv = None
    for ps in range(NPASS):
        if ps > 0:
            q8 = _q_call(e8, bd)
        q = q8.reshape(E, DE)
        if ps == 0:
            edge, s2, cnt2 = _get_sc(True)(src2, dst2, p, pd, q)
        else:
            edge, s2 = _get_sc(False)(src2, dst2, p, pd, q)
        e8 = edge.reshape(E // 8, 8 * DE)
        s2r = s2.reshape(NC, NP * DE // 128, 128)
        if ps == 0:
            c2r = cnt2.reshape(NC, NP * DE // 128, 128)
            na, u, p, pd, inv = _node1_call(na, s2r, c2r, br, b2, u, w_mid)
        elif ps < NPASS - 1:
            na, u, p, pd = _node2_call(na, s2r, inv, br, b2, u, w_mid)
        else:
            return _node3_call(na, s2r, inv, br, b2, u, w_fin)
